# Initial kernel scaffold; baseline (speedup 1.0000x reference)
import jax
import jax.numpy as jnp
from jax.experimental import pallas as pl


def kernel(features, edge_index, W1, att_src1, att_dst1, W12, att_src12, att_dst12, W2, att_src2, att_dst2, W2h):
    raise NotImplementedError("write your pallas kernel here")


# speedup vs baseline: 23.4026x; 23.4026x over previous
import jax
import jax.numpy as jnp
from jax import lax
from jax.experimental import pallas as pl
from jax.experimental.pallas import tpu as pltpu
from jax.experimental.pallas import tpu_sc as plsc

N = 10000
NP = 10240
E = 320000
NC, NS = 2, 16
NW = NC * NS
EPT = E // NW
CH = 80
NCH = EPT // CH
RPT = NP // NS
LANES = 16
SLOPE = 0.2


def _sc_attn_call(D, mode):
    mesh = plsc.VectorSubcoreMesh(core_axis_name="c", subcore_axis_name="s",
                                  num_cores=NC, num_subcores=NS)
    compute_ex = mode in ("first", "mid")

    out_type = [jax.ShapeDtypeStruct((NC, NP, D), jnp.float32)]
    if compute_ex:
        out_type.append(jax.ShapeDtypeStruct((NC, 1, NP), jnp.float32))
    if mode == "first":
        out_type.append(jax.ShapeDtypeStruct((E,), jnp.float32))

    scratch = dict(
        src2_v=pltpu.VMEM((NCH, CH), jnp.int32),
        dst2_v=pltpu.VMEM((NCH, CH), jnp.int32),
        exc_v=pltpu.VMEM((CH,), jnp.float32),
        asg_v=pltpu.VMEM((CH,), jnp.float32),
        adg_v=pltpu.VMEM((CH,), jnp.float32),
        rows_v=pltpu.VMEM((CH, D), jnp.float32),
        sem=pltpu.SemaphoreType.DMA,
        acc_s=pltpu.VMEM_SHARED((NP, D), jnp.float32),
        den_s=pltpu.VMEM_SHARED((NP,), jnp.float32),
    )
    scratch_types = list(scratch.values())
    names = list(scratch.keys())

    def body(*refs):
        if compute_ex:
            n_in = 7
            xp_h, as_h, ad_h, src2_h, dst2_h, zero_h, z1_h = refs[:n_in]
            ex_io = None
        else:
            n_in = 6
            xp_h, src2_h, dst2_h, zero_h, z1_h, ex_io = refs[:n_in]
        outs = refs[n_in:n_in + len(out_type)]
        scr = dict(zip(names, refs[n_in + len(out_type):]))
        acc_h = outs[0]
        if mode == "first":
            ex_io = outs[2]

        cid = lax.axis_index("c")
        sid = lax.axis_index("s")
        wid = sid * NC + cid
        ebase = wid * EPT

        pltpu.sync_copy(zero_h, scr["acc_s"].at[pl.ds(sid * RPT, RPT)])
        pltpu.sync_copy(z1_h, scr["den_s"].at[pl.ds(sid * RPT, RPT)])
        pltpu.sync_copy(src2_h.at[wid], scr["src2_v"])
        pltpu.sync_copy(dst2_h.at[wid], scr["dst2_v"])

        plsc.subcore_barrier()

        def chunk(j, c):
            sidx = scr["src2_v"].at[j]
            didx = scr["dst2_v"].at[j]
            if compute_ex:
                cp_a = pltpu.async_copy(as_h.at[sidx], scr["asg_v"],
                                        scr["sem"])
                cp_b = pltpu.async_copy(ad_h.at[didx], scr["adg_v"],
                                        scr["sem"])
            cp_r = pltpu.async_copy(xp_h.at[sidx], scr["rows_v"], scr["sem"])
            if compute_ex:
                cp_a.wait()
                cp_b.wait()
                for g in range(CH // LANES):
                    sl = pl.ds(g * LANES, LANES)
                    a = scr["asg_v"][sl] + scr["adg_v"][sl]
                    e = jnp.where(a > 0, a, SLOPE * a)
                    scr["exc_v"][sl] = jnp.exp(e)
                pltpu.sync_copy(scr["exc_v"], scr["den_s"].at[didx],
                                add=True)
                if mode == "first":
                    pltpu.sync_copy(scr["exc_v"],
                                    ex_io.at[pl.ds(ebase + j * CH, CH)])
            else:
                pltpu.sync_copy(ex_io.at[pl.ds(ebase + j * CH, CH)],
                                scr["exc_v"])
            cp_r.wait()

            def scale(e2, c2):
                exb = plsc.load_gather(
                    scr["exc_v"], [jnp.full((LANES,), e2, jnp.int32)])
                for k in range(D // LANES):
                    sl = pl.ds(k * LANES, LANES)
                    scr["rows_v"][e2, sl] = scr["rows_v"][e2, sl] * exb
                return c2
            lax.fori_loop(0, CH, scale, 0)
            pltpu.sync_copy(scr["rows_v"], scr["acc_s"].at[didx], add=True)
            return c
        lax.fori_loop(0, NCH, chunk, 0)

        plsc.subcore_barrier()
        pltpu.sync_copy(scr["acc_s"].at[pl.ds(sid * RPT, RPT)],
                        acc_h.at[cid, pl.ds(sid * RPT, RPT)])
        if compute_ex:
            pltpu.sync_copy(scr["den_s"].at[pl.ds(sid * RPT, RPT)],
                            outs[1].at[cid, 0, pl.ds(sid * RPT, RPT)])

    return pl.kernel(body, out_type=tuple(out_type), mesh=mesh,
                     scratch_types=scratch_types, name=f"sc_attn_{mode}_{D}",
                     compiler_params=pltpu.CompilerParams(
                         needs_layout_passes=False,
                         use_tc_tiling_on_sc=False))


_BLK = 1000
_GRID = N // _BLK
_BLKP = 1024
_GRIDP = NP // _BLKP


def _combine(acc, den, elu):
    den_sum = jnp.sum(den, axis=1) + 1e-16
    o = (acc[0] + acc[1]) / den_sum[:, None]
    if elu:
        o = jnp.where(o > 0, o, jnp.exp(jnp.minimum(o, 0.0)) - 1.0)
    return o


def _tc_first(feats, W, att_s, att_d):
    Din, K = W.shape

    def body(x_ref, w_ref, s_ref, d_ref, xp_ref, os_ref, od_ref):
        xp = jnp.dot(x_ref[...], w_ref[...],
                     preferred_element_type=jnp.float32)
        xp_ref[...] = xp
        os_ref[...] = jnp.sum(xp * s_ref[...], axis=1).reshape(1, 1, _BLK)
        od_ref[...] = jnp.sum(xp * d_ref[...], axis=1).reshape(1, 1, _BLK)

    return pl.pallas_call(
        body,
        grid=(_GRID,),
        in_specs=[pl.BlockSpec((_BLK, Din), lambda i: (i, 0)),
                  pl.BlockSpec((Din, K), lambda i: (0, 0)),
                  pl.BlockSpec((1, K), lambda i: (0, 0)),
                  pl.BlockSpec((1, K), lambda i: (0, 0))],
        out_specs=[pl.BlockSpec((_BLK, K), lambda i: (i, 0)),
                   pl.BlockSpec((1, 1, _BLK), lambda i: (i, 0, 0)),
                   pl.BlockSpec((1, 1, _BLK), lambda i: (i, 0, 0))],
        out_shape=[jax.ShapeDtypeStruct((N, K), jnp.float32),
                   jax.ShapeDtypeStruct((_GRID, 1, _BLK), jnp.float32),
                   jax.ShapeDtypeStruct((_GRID, 1, _BLK), jnp.float32)],
    )(feats, W, att_s, att_d)


def _tc_mid(acc, den, W, att_s, att_d):
    D = acc.shape[-1]
    K = W.shape[1]

    def body(a_ref, n_ref, w_ref, s_ref, d_ref, xp_ref, os_ref, od_ref):
        h = _combine(a_ref[...], n_ref[...], elu=True)
        xp = jnp.dot(h, w_ref[...], preferred_element_type=jnp.float32)
        xp_ref[...] = xp
        os_ref[...] = jnp.sum(xp * s_ref[...], axis=1).reshape(1, 1, _BLKP)
        od_ref[...] = jnp.sum(xp * d_ref[...], axis=1).reshape(1, 1, _BLKP)

    return pl.pallas_call(
        body,
        grid=(_GRIDP,),
        in_specs=[pl.BlockSpec((NC, _BLKP, D), lambda i: (0, i, 0)),
                  pl.BlockSpec((_BLKP, NC), lambda i: (i, 0)),
                  pl.BlockSpec((D, K), lambda i: (0, 0)),
                  pl.BlockSpec((1, K), lambda i: (0, 0)),
                  pl.BlockSpec((1, K), lambda i: (0, 0))],
        out_specs=[pl.BlockSpec((_BLKP, K), lambda i: (i, 0)),
                   pl.BlockSpec((1, 1, _BLKP), lambda i: (i, 0, 0)),
                   pl.BlockSpec((1, 1, _BLKP), lambda i: (i, 0, 0))],
        out_shape=[jax.ShapeDtypeStruct((NP, K), jnp.float32),
                   jax.ShapeDtypeStruct((_GRIDP, 1, _BLKP), jnp.float32),
                   jax.ShapeDtypeStruct((_GRIDP, 1, _BLKP), jnp.float32)],
    )(acc, den, W, att_s, att_d)


def _tc_l4(acc, den, W2T, W2h):
    D = acc.shape[-1]
    K = W2T.shape[1]

    def body(a_ref, n_ref, wt_ref, wh_ref, h2_ref, xp3_ref, hh_ref):
        h2 = _combine(a_ref[...], n_ref[...], elu=False)
        h2_ref[...] = h2
        xp3_ref[...] = jnp.dot(h2, wt_ref[...],
                               preferred_element_type=jnp.float32)
        hh_ref[...] = jnp.dot(h2, wh_ref[...],
                              preferred_element_type=jnp.float32)

    return pl.pallas_call(
        body,
        grid=(_GRIDP,),
        in_specs=[pl.BlockSpec((NC, _BLKP, D), lambda i: (0, i, 0)),
                  pl.BlockSpec((_BLKP, NC), lambda i: (i, 0)),
                  pl.BlockSpec((D, K), lambda i: (0, 0)),
                  pl.BlockSpec((D, D), lambda i: (0, 0))],
        out_specs=[pl.BlockSpec((_BLKP, D), lambda i: (i, 0)),
                   pl.BlockSpec((_BLKP, K), lambda i: (i, 0)),
                   pl.BlockSpec((_BLKP, D), lambda i: (i, 0))],
        out_shape=[jax.ShapeDtypeStruct((NP, D), jnp.float32),
                   jax.ShapeDtypeStruct((NP, K), jnp.float32),
                   jax.ShapeDtypeStruct((NP, D), jnp.float32)],
    )(acc, den, W2T, W2h)


def _tc_final(acc, den, W1T):
    D = acc.shape[-1]
    K = W1T.shape[1]

    def body(a_ref, n_ref, w_ref, h4_ref):
        h3 = _combine(a_ref[...], n_ref[...], elu=True)
        h4_ref[...] = jnp.dot(h3, w_ref[...],
                              preferred_element_type=jnp.float32)

    return pl.pallas_call(
        body,
        grid=(_GRIDP,),
        in_specs=[pl.BlockSpec((NC, _BLKP, D), lambda i: (0, i, 0)),
                  pl.BlockSpec((_BLKP, NC), lambda i: (i, 0)),
                  pl.BlockSpec((D, K), lambda i: (0, 0))],
        out_specs=pl.BlockSpec((_BLKP, K), lambda i: (i, 0)),
        out_shape=jax.ShapeDtypeStruct((NP, K), jnp.float32),
    )(acc, den, W1T)


def kernel(features, edge_index, W1, att_src1, att_dst1, W12, att_src12,
           att_dst12, W2, att_src2, att_dst2, W2h):
    src = edge_index[0]
    dst = edge_index[1]
    src2 = src.reshape(NW, NCH, CH)
    dst2 = dst.reshape(NW, NCH, CH)
    z128 = jnp.zeros((RPT, 128), jnp.float32)
    z64 = jnp.zeros((RPT, 64), jnp.float32)
    z1 = jnp.zeros((RPT,), jnp.float32)
    exdummy = jnp.zeros((8,), jnp.float32)

    sc_first = _sc_attn_call(128, "first")
    sc_mid128 = _sc_attn_call(128, "mid")
    sc_mid64 = _sc_attn_call(64, "mid")
    sc_reuse = _sc_attn_call(128, "reuse")

    xp1, a1s, a1d = _tc_first(features, W1, att_src1[None, :],
                              att_dst1[None, :])
    acc1, den1, ex1 = sc_first(xp1, a1s.reshape(N), a1d.reshape(N),
                               src2, dst2, z128, z1)
    den1t = den1.reshape(NC, NP).T
    xp12, a12s, a12d = _tc_mid(acc1, den1t, W12, att_src12[None, :],
                               att_dst12[None, :])
    acc12, den12 = sc_mid128(xp12, a12s.reshape(NP), a12d.reshape(NP),
                             src2, dst2, z128, z1)
    xp2, a2s, a2d = _tc_mid(acc12, den12.reshape(NC, NP).T, W2,
                            att_src2[None, :], att_dst2[None, :])
    acc2, den2 = sc_mid64(xp2, a2s.reshape(NP), a2d.reshape(NP),
                          src2, dst2, z64, z1)
    h2, xp3, h_h = _tc_l4(acc2, den2.reshape(NC, NP).T, W2.T, W2h)
    (acc3,) = sc_reuse(xp3, src2, dst2, z128, z1, ex1)
    h4 = _tc_final(acc3, den1t, W1.T)
    return (h2[:N], h4[:N], h_h[:N])


# speedup vs baseline: 37.6401x; 1.6084x over previous
import jax
import jax.numpy as jnp
from jax import lax
from jax.experimental import pallas as pl
from jax.experimental.pallas import tpu as pltpu
from jax.experimental.pallas import tpu_sc as plsc

N = 10000
NP = 10240
E = 320000
NC, NS = 2, 16
NW = NC * NS
EPT = E // NW
CH = 80
NCH = EPT // CH
RPT = NP // NS
LANES = 16
SLOPE = 0.2


def _sc_attn_call(D, mode):
    mesh = plsc.VectorSubcoreMesh(core_axis_name="c", subcore_axis_name="s",
                                  num_cores=NC, num_subcores=NS)
    compute_ex = mode in ("first", "mid")

    out_type = [jax.ShapeDtypeStruct((NC, NP, D), jnp.float32)]
    if compute_ex:
        out_type.append(jax.ShapeDtypeStruct((NC, 1, NP), jnp.float32))
    if mode == "first":
        out_type.append(jax.ShapeDtypeStruct((E,), jnp.float32))

    scratch = dict(
        src2_v=pltpu.VMEM((NCH, CH), jnp.int32),
        dst2_v=pltpu.VMEM((NCH, CH), jnp.int32),
        exc0=pltpu.VMEM((CH,), jnp.float32),
        exc1=pltpu.VMEM((CH,), jnp.float32),
        asg0=pltpu.VMEM((CH,), jnp.float32),
        asg1=pltpu.VMEM((CH,), jnp.float32),
        adg0=pltpu.VMEM((CH,), jnp.float32),
        adg1=pltpu.VMEM((CH,), jnp.float32),
        rows0=pltpu.VMEM((CH, D), jnp.float32),
        rows1=pltpu.VMEM((CH, D), jnp.float32),
        semR=pltpu.SemaphoreType.DMA,
        semA=pltpu.SemaphoreType.DMA,
        semB=pltpu.SemaphoreType.DMA,
        semX=pltpu.SemaphoreType.DMA,
        acc_s=pltpu.VMEM_SHARED((NP, D), jnp.float32),
        den_s=pltpu.VMEM_SHARED((NP,), jnp.float32),
    )
    scratch_types = list(scratch.values())
    names = list(scratch.keys())

    def body(*refs):
        if compute_ex:
            n_in = 7
            xp_h, as_h, ad_h, src2_h, dst2_h, zero_h, z1_h = refs[:n_in]
            ex_io = None
        else:
            n_in = 6
            xp_h, src2_h, dst2_h, zero_h, z1_h, ex_io = refs[:n_in]
        outs = refs[n_in:n_in + len(out_type)]
        scr = dict(zip(names, refs[n_in + len(out_type):]))
        acc_h = outs[0]
        if mode == "first":
            ex_io = outs[2]

        cid = lax.axis_index("c")
        sid = lax.axis_index("s")
        wid = sid * NC + cid
        ebase = wid * EPT

        pltpu.sync_copy(zero_h, scr["acc_s"].at[pl.ds(sid * RPT, RPT)])
        pltpu.sync_copy(z1_h, scr["den_s"].at[pl.ds(sid * RPT, RPT)])
        pltpu.sync_copy(src2_h.at[wid], scr["src2_v"])
        pltpu.sync_copy(dst2_h.at[wid], scr["dst2_v"])

        plsc.subcore_barrier()

        def fire(j, p):
            sidx = scr["src2_v"].at[j]
            pltpu.async_copy(xp_h.at[sidx], scr[f"rows{p}"], scr["semR"])
            if compute_ex:
                didx = scr["dst2_v"].at[j]
                pltpu.async_copy(as_h.at[sidx], scr[f"asg{p}"], scr["semA"])
                pltpu.async_copy(ad_h.at[didx], scr[f"adg{p}"], scr["semB"])
            else:
                pltpu.async_copy(ex_io.at[pl.ds(ebase + j * CH, CH)],
                                 scr[f"exc{p}"], scr["semX"])

        def process(j, p):
            didx = scr["dst2_v"].at[j]
            if compute_ex:
                pltpu.make_async_copy(as_h.at[pl.ds(0, CH)], scr[f"asg{p}"],
                                      scr["semA"]).wait()
                pltpu.make_async_copy(ad_h.at[pl.ds(0, CH)], scr[f"adg{p}"],
                                      scr["semB"]).wait()
                for g in range(CH // LANES):
                    sl = pl.ds(g * LANES, LANES)
                    a = scr[f"asg{p}"][sl] + scr[f"adg{p}"][sl]
                    e = jnp.where(a > 0, a, SLOPE * a)
                    scr[f"exc{p}"][sl] = jnp.exp(e)
                pltpu.sync_copy(scr[f"exc{p}"], scr["den_s"].at[didx],
                                add=True)
                if mode == "first":
                    pltpu.sync_copy(scr[f"exc{p}"],
                                    ex_io.at[pl.ds(ebase + j * CH, CH)])
            else:
                pltpu.make_async_copy(ex_io.at[pl.ds(0, CH)], scr[f"exc{p}"],
                                      scr["semX"]).wait()
            pltpu.make_async_copy(xp_h.at[pl.ds(0, CH)], scr[f"rows{p}"],
                                  scr["semR"]).wait()

            def scale(e2, c2):
                exb = plsc.load_gather(
                    scr[f"exc{p}"], [jnp.full((LANES,), e2, jnp.int32)])
                for k in range(D // LANES):
                    sl = pl.ds(k * LANES, LANES)
                    scr[f"rows{p}"][e2, sl] = scr[f"rows{p}"][e2, sl] * exb
                return c2
            lax.fori_loop(0, CH, scale, 0)
            pltpu.sync_copy(scr[f"rows{p}"], scr["acc_s"].at[didx], add=True)

        fire(0, 0)

        def pair(j2, c):
            j = j2 * 2
            fire(j + 1, 1)
            process(j, 0)
            fire(j + 2, 0)
            process(j + 1, 1)
            return c
        lax.fori_loop(0, (NCH - 1) // 2, pair, 0)
        process(NCH - 1, 0)

        plsc.subcore_barrier()
        pltpu.sync_copy(scr["acc_s"].at[pl.ds(sid * RPT, RPT)],
                        acc_h.at[cid, pl.ds(sid * RPT, RPT)])
        if compute_ex:
            pltpu.sync_copy(scr["den_s"].at[pl.ds(sid * RPT, RPT)],
                            outs[1].at[cid, 0, pl.ds(sid * RPT, RPT)])

    return pl.kernel(body, out_type=tuple(out_type), mesh=mesh,
                     scratch_types=scratch_types, name=f"sc_attn_{mode}_{D}",
                     compiler_params=pltpu.CompilerParams(
                         needs_layout_passes=False,
                         use_tc_tiling_on_sc=False))


_BLK = 1000
_GRID = N // _BLK
_BLKP = 1024
_GRIDP = NP // _BLKP


def _combine(acc, den, elu):
    den_sum = jnp.sum(den, axis=1) + 1e-16
    o = (acc[0] + acc[1]) / den_sum[:, None]
    if elu:
        o = jnp.where(o > 0, o, jnp.exp(jnp.minimum(o, 0.0)) - 1.0)
    return o


def _tc_first(feats, W, att_s, att_d):
    Din, K = W.shape

    def body(x_ref, w_ref, s_ref, d_ref, xp_ref, os_ref, od_ref):
        xp = jnp.dot(x_ref[...], w_ref[...],
                     preferred_element_type=jnp.float32)
        xp_ref[...] = xp
        os_ref[...] = jnp.sum(xp * s_ref[...], axis=1).reshape(1, 1, _BLK)
        od_ref[...] = jnp.sum(xp * d_ref[...], axis=1).reshape(1, 1, _BLK)

    return pl.pallas_call(
        body,
        grid=(_GRID,),
        in_specs=[pl.BlockSpec((_BLK, Din), lambda i: (i, 0)),
                  pl.BlockSpec((Din, K), lambda i: (0, 0)),
                  pl.BlockSpec((1, K), lambda i: (0, 0)),
                  pl.BlockSpec((1, K), lambda i: (0, 0))],
        out_specs=[pl.BlockSpec((_BLK, K), lambda i: (i, 0)),
                   pl.BlockSpec((1, 1, _BLK), lambda i: (i, 0, 0)),
                   pl.BlockSpec((1, 1, _BLK), lambda i: (i, 0, 0))],
        out_shape=[jax.ShapeDtypeStruct((N, K), jnp.float32),
                   jax.ShapeDtypeStruct((_GRID, 1, _BLK), jnp.float32),
                   jax.ShapeDtypeStruct((_GRID, 1, _BLK), jnp.float32)],
    )(feats, W, att_s, att_d)


def _tc_mid(acc, den, W, att_s, att_d):
    D = acc.shape[-1]
    K = W.shape[1]

    def body(a_ref, n_ref, w_ref, s_ref, d_ref, xp_ref, os_ref, od_ref):
        h = _combine(a_ref[...], n_ref[...], elu=True)
        xp = jnp.dot(h, w_ref[...], preferred_element_type=jnp.float32)
        xp_ref[...] = xp
        os_ref[...] = jnp.sum(xp * s_ref[...], axis=1).reshape(1, 1, _BLKP)
        od_ref[...] = jnp.sum(xp * d_ref[...], axis=1).reshape(1, 1, _BLKP)

    return pl.pallas_call(
        body,
        grid=(_GRIDP,),
        in_specs=[pl.BlockSpec((NC, _BLKP, D), lambda i: (0, i, 0)),
                  pl.BlockSpec((_BLKP, NC), lambda i: (i, 0)),
                  pl.BlockSpec((D, K), lambda i: (0, 0)),
                  pl.BlockSpec((1, K), lambda i: (0, 0)),
                  pl.BlockSpec((1, K), lambda i: (0, 0))],
        out_specs=[pl.BlockSpec((_BLKP, K), lambda i: (i, 0)),
                   pl.BlockSpec((1, 1, _BLKP), lambda i: (i, 0, 0)),
                   pl.BlockSpec((1, 1, _BLKP), lambda i: (i, 0, 0))],
        out_shape=[jax.ShapeDtypeStruct((NP, K), jnp.float32),
                   jax.ShapeDtypeStruct((_GRIDP, 1, _BLKP), jnp.float32),
                   jax.ShapeDtypeStruct((_GRIDP, 1, _BLKP), jnp.float32)],
    )(acc, den, W, att_s, att_d)


def _tc_l4(acc, den, W2T, W2h):
    D = acc.shape[-1]
    K = W2T.shape[1]

    def body(a_ref, n_ref, wt_ref, wh_ref, h2_ref, xp3_ref, hh_ref):
        h2 = _combine(a_ref[...], n_ref[...], elu=False)
        h2_ref[...] = h2
        xp3_ref[...] = jnp.dot(h2, wt_ref[...],
                               preferred_element_type=jnp.float32)
        hh_ref[...] = jnp.dot(h2, wh_ref[...],
                              preferred_element_type=jnp.float32)

    return pl.pallas_call(
        body,
        grid=(_GRIDP,),
        in_specs=[pl.BlockSpec((NC, _BLKP, D), lambda i: (0, i, 0)),
                  pl.BlockSpec((_BLKP, NC), lambda i: (i, 0)),
                  pl.BlockSpec((D, K), lambda i: (0, 0)),
                  pl.BlockSpec((D, D), lambda i: (0, 0))],
        out_specs=[pl.BlockSpec((_BLKP, D), lambda i: (i, 0)),
                   pl.BlockSpec((_BLKP, K), lambda i: (i, 0)),
                   pl.BlockSpec((_BLKP, D), lambda i: (i, 0))],
        out_shape=[jax.ShapeDtypeStruct((NP, D), jnp.float32),
                   jax.ShapeDtypeStruct((NP, K), jnp.float32),
                   jax.ShapeDtypeStruct((NP, D), jnp.float32)],
    )(acc, den, W2T, W2h)


def _tc_final(acc, den, W1T):
    D = acc.shape[-1]
    K = W1T.shape[1]

    def body(a_ref, n_ref, w_ref, h4_ref):
        h3 = _combine(a_ref[...], n_ref[...], elu=True)
        h4_ref[...] = jnp.dot(h3, w_ref[...],
                              preferred_element_type=jnp.float32)

    return pl.pallas_call(
        body,
        grid=(_GRIDP,),
        in_specs=[pl.BlockSpec((NC, _BLKP, D), lambda i: (0, i, 0)),
                  pl.BlockSpec((_BLKP, NC), lambda i: (i, 0)),
                  pl.BlockSpec((D, K), lambda i: (0, 0))],
        out_specs=pl.BlockSpec((_BLKP, K), lambda i: (i, 0)),
        out_shape=jax.ShapeDtypeStruct((NP, K), jnp.float32),
    )(acc, den, W1T)


def kernel(features, edge_index, W1, att_src1, att_dst1, W12, att_src12,
           att_dst12, W2, att_src2, att_dst2, W2h):
    src = edge_index[0]
    dst = edge_index[1]
    src2 = src.reshape(NW, NCH, CH)
    dst2 = dst.reshape(NW, NCH, CH)
    z128 = jnp.zeros((RPT, 128), jnp.float32)
    z64 = jnp.zeros((RPT, 64), jnp.float32)
    z1 = jnp.zeros((RPT,), jnp.float32)
    exdummy = jnp.zeros((8,), jnp.float32)

    sc_first = _sc_attn_call(128, "first")
    sc_mid128 = _sc_attn_call(128, "mid")
    sc_mid64 = _sc_attn_call(64, "mid")
    sc_reuse = _sc_attn_call(128, "reuse")

    xp1, a1s, a1d = _tc_first(features, W1, att_src1[None, :],
                              att_dst1[None, :])
    acc1, den1, ex1 = sc_first(xp1, a1s.reshape(N), a1d.reshape(N),
                               src2, dst2, z128, z1)
    den1t = den1.reshape(NC, NP).T
    xp12, a12s, a12d = _tc_mid(acc1, den1t, W12, att_src12[None, :],
                               att_dst12[None, :])
    acc12, den12 = sc_mid128(xp12, a12s.reshape(NP), a12d.reshape(NP),
                             src2, dst2, z128, z1)
    xp2, a2s, a2d = _tc_mid(acc12, den12.reshape(NC, NP).T, W2,
                            att_src2[None, :], att_dst2[None, :])
    acc2, den2 = sc_mid64(xp2, a2s.reshape(NP), a2d.reshape(NP),
                          src2, dst2, z64, z1)
    h2, xp3, h_h = _tc_l4(acc2, den2.reshape(NC, NP).T, W2.T, W2h)
    (acc3,) = sc_reuse(xp3, src2, dst2, z128, z1, ex1)
    h4 = _tc_final(acc3, den1t, W1.T)
    return (h2[:N], h4[:N], h_h[:N])
